# baseline (speedup 1.0000x reference)
import functools

import jax
import jax.numpy as jnp
from jax import lax
from jax.experimental import pallas as pl
from jax.experimental.pallas import tpu as pltpu
from jax.experimental.pallas import tpu_sc as plsc

NSEG = 500
SEG_PAD = 512
ACC_STRIDE = 513
KSETS = 5
NC = 2
NS = 16
NW = NC * NS
LANES = 16

N_TOTAL = 6400000
PER_TILE = N_TOTAL // NW
CHUNK = 20000
NCHUNKS = PER_TILE // CHUNK
UNROLL = 10
ITERS = CHUNK // (LANES * UNROLL)


def _make_sc_kernel():
  mesh = plsc.VectorSubcoreMesh(core_axis_name="c", subcore_axis_name="s")

  @functools.partial(
      pl.kernel,
      out_type=jax.ShapeDtypeStruct((NW, SEG_PAD), jnp.float32),
      mesh=mesh,
      compiler_params=pltpu.CompilerParams(needs_layout_passes=False),
      scratch_types=[
          pltpu.VMEM((CHUNK,), jnp.float32),
          pltpu.VMEM((CHUNK,), jnp.float32),
          pltpu.VMEM((CHUNK,), jnp.int32),
          pltpu.VMEM((CHUNK,), jnp.int32),
          pltpu.VMEM((KSETS * LANES * ACC_STRIDE,), jnp.float32),
          pltpu.VMEM((SEG_PAD,), jnp.float32),
          pltpu.SemaphoreType.DMA,
          pltpu.SemaphoreType.DMA,
      ],
  )
  def seg_sum(in_hbm, lab_hbm, out_hbm, in0, in1, lb0, lb1, acc, part,
              sem0, sem1):
    cid = lax.axis_index("c")
    sid = lax.axis_index("s")
    wid = sid * NC + cid
    base = wid * PER_TILE

    lane_iota = lax.iota(jnp.int32, LANES)
    row_bases = [(lane_iota + k * LANES) * ACC_STRIDE for k in range(KSETS)]
    zeros16 = jnp.zeros((LANES,), jnp.float32)

    def zero_body(cb, carry):
      acc[pl.ds(cb * LANES, LANES)] = zeros16
      return carry

    lax.fori_loop(0, (KSETS * LANES * ACC_STRIDE) // LANES, zero_body, 0)

    bufs = ((in0, lb0, sem0), (in1, lb1, sem1))

    def start(g):
      inb, lbb, sem = bufs[g % 2]
      off = base + g * CHUNK
      h1 = pltpu.async_copy(in_hbm.at[pl.ds(off, CHUNK)], inb, sem)
      h2 = pltpu.async_copy(lab_hbm.at[pl.ds(off, CHUNK)], lbb, sem)
      return h1, h2

    handles = start(0)
    for g in range(NCHUNKS):
      nxt = start(g + 1) if g + 1 < NCHUNKS else None
      handles[0].wait()
      handles[1].wait()
      inb, lbb, _ = bufs[g % 2]

      def chunk_body(i, carry, inb=inb, lbb=lbb):
        for u in range(UNROLL):
          off = (i * UNROLL + u) * LANES
          lab = lbb[pl.ds(off, LANES)]
          val = inb[pl.ds(off, LANES)]
          plsc.addupdate_scatter(acc, [row_bases[u % KSETS] + lab], val)
        return carry

      lax.fori_loop(0, ITERS, chunk_body, 0)
      handles = nxt

    def reduce_body(cb, carry):
      s = zeros16
      for r in range(KSETS * LANES):
        s = s + acc[pl.ds(r * ACC_STRIDE + cb * LANES, LANES)]
      part[pl.ds(cb * LANES, LANES)] = s
      return carry

    lax.fori_loop(0, SEG_PAD // LANES, reduce_body, 0)
    pltpu.sync_copy(part, out_hbm.at[wid])

  return seg_sum


_SEG_SUM = _make_sc_kernel()


@jax.jit
def kernel(inputs, labels):
  partials = _SEG_SUM(inputs, labels)
  return jnp.sum(partials, axis=0)[:NSEG]


# speedup vs baseline: 2.2037x; 2.2037x over previous
import functools

import jax
import jax.numpy as jnp
from jax import lax
from jax.experimental import pallas as pl
from jax.experimental.pallas import tpu as pltpu
from jax.experimental.pallas import tpu_sc as plsc

NSEG = 500
SEG_PAD = 512
ACC_STRIDE = 513
NC = 2
NS = 16
NW = NC * NS
LANES = 16

N_TOTAL = 6400000
PER_TILE = N_TOTAL // NW
CHUNK = 20000
NCHUNKS = PER_TILE // CHUNK
UNROLL = 10


def _make_sc_kernel():
  mesh = plsc.VectorSubcoreMesh(core_axis_name="c", subcore_axis_name="s")

  @functools.partial(
      pl.kernel,
      out_type=jax.ShapeDtypeStruct((NW, SEG_PAD), jnp.float32),
      mesh=mesh,
      compiler_params=pltpu.CompilerParams(needs_layout_passes=False),
      scratch_types=[
          pltpu.VMEM((CHUNK,), jnp.float32),
          pltpu.VMEM((CHUNK,), jnp.float32),
          pltpu.VMEM((CHUNK,), jnp.int32),
          pltpu.VMEM((CHUNK,), jnp.int32),
          pltpu.VMEM((LANES * ACC_STRIDE,), jnp.float32),
          pltpu.VMEM((SEG_PAD,), jnp.float32),
          pltpu.SemaphoreType.DMA,
          pltpu.SemaphoreType.DMA,
      ],
  )
  def seg_sum(in_hbm, lab_hbm, out_hbm, in0, in1, lb0, lb1, acc, part,
              sem0, sem1):
    cid = lax.axis_index("c")
    sid = lax.axis_index("s")
    wid = sid * NC + cid
    base = wid * PER_TILE

    row_base = lax.iota(jnp.int32, LANES) * ACC_STRIDE
    zeros16 = jnp.zeros((LANES,), jnp.float32)

    @plsc.parallel_loop(0, (LANES * ACC_STRIDE) // LANES, unroll=8)
    def _(cb):
      acc[pl.ds(cb * LANES, LANES)] = zeros16

    bufs = ((in0, lb0, sem0), (in1, lb1, sem1))

    def start(g):
      inb, lbb, sem = bufs[g % 2]
      off = base + g * CHUNK
      h1 = pltpu.async_copy(in_hbm.at[pl.ds(off, CHUNK)], inb, sem)
      h2 = pltpu.async_copy(lab_hbm.at[pl.ds(off, CHUNK)], lbb, sem)
      return h1, h2

    handles = start(0)
    for g in range(NCHUNKS):
      nxt = start(g + 1) if g + 1 < NCHUNKS else None
      handles[0].wait()
      handles[1].wait()
      inb, lbb, _ = bufs[g % 2]

      @plsc.parallel_loop(0, CHUNK // LANES, unroll=UNROLL)
      def _(i, inb=inb, lbb=lbb):
        off = i * LANES
        lab = lbb[pl.ds(off, LANES)]
        val = inb[pl.ds(off, LANES)]
        plsc.addupdate_scatter(acc, [row_base + lab], val)

      handles = nxt

    def reduce_body(cb, carry):
      s = zeros16
      for r in range(LANES):
        s = s + acc[pl.ds(r * ACC_STRIDE + cb * LANES, LANES)]
      part[pl.ds(cb * LANES, LANES)] = s
      return carry

    lax.fori_loop(0, SEG_PAD // LANES, reduce_body, 0)
    pltpu.sync_copy(part, out_hbm.at[wid])

  return seg_sum


_SEG_SUM = _make_sc_kernel()


@jax.jit
def kernel(inputs, labels):
  partials = _SEG_SUM(inputs, labels)
  return jnp.sum(partials, axis=0)[:NSEG]


# speedup vs baseline: 2.2468x; 1.0196x over previous
import functools

import jax
import jax.numpy as jnp
from jax import lax
from jax.experimental import pallas as pl
from jax.experimental.pallas import tpu as pltpu
from jax.experimental.pallas import tpu_sc as plsc

NSEG = 500
SEG_PAD = 512
ACC_STRIDE = 513
NC = 2
NS = 16
NW = NC * NS
LANES = 16

N_TOTAL = 6400000
PER_TILE = N_TOTAL // NW
CHUNK = 20000
NCHUNKS = PER_TILE // CHUNK
UNROLL = 10


def _make_sc_kernel():
  mesh = plsc.VectorSubcoreMesh(core_axis_name="c", subcore_axis_name="s")

  @functools.partial(
      pl.kernel,
      out_type=jax.ShapeDtypeStruct((NW, SEG_PAD), jnp.float32),
      mesh=mesh,
      compiler_params=pltpu.CompilerParams(needs_layout_passes=False),
      scratch_types=[
          pltpu.VMEM((CHUNK,), jnp.float32),
          pltpu.VMEM((CHUNK,), jnp.float32),
          pltpu.VMEM((CHUNK,), jnp.int32),
          pltpu.VMEM((CHUNK,), jnp.int32),
          pltpu.VMEM((LANES * ACC_STRIDE,), jnp.float32),
          pltpu.VMEM((SEG_PAD,), jnp.float32),
          pltpu.SemaphoreType.DMA,
          pltpu.SemaphoreType.DMA,
      ],
  )
  def seg_sum(in_hbm, lab_hbm, out_hbm, in0, in1, lb0, lb1, acc, part,
              sem0, sem1):
    cid = lax.axis_index("c")
    sid = lax.axis_index("s")
    wid = sid * NC + cid
    base = wid * PER_TILE

    row_base = lax.iota(jnp.int32, LANES) * ACC_STRIDE
    zeros16 = jnp.zeros((LANES,), jnp.float32)

    @plsc.parallel_loop(0, (LANES * ACC_STRIDE) // LANES, unroll=8)
    def _(cb):
      acc[pl.ds(cb * LANES, LANES)] = zeros16

    bufs = ((in0, lb0, sem0), (in1, lb1, sem1))

    def start(g, sl):
      inb, lbb, sem = bufs[sl]
      off = base + g * CHUNK
      pltpu.async_copy(in_hbm.at[pl.ds(off, CHUNK)], inb, sem)
      pltpu.async_copy(lab_hbm.at[pl.ds(off, CHUNK)], lbb, sem)

    def drain(sl):
      inb, lbb, sem = bufs[sl]
      pltpu.make_async_copy(in_hbm.at[pl.ds(0, CHUNK)], inb, sem).wait()
      pltpu.make_async_copy(lab_hbm.at[pl.ds(0, CHUNK)], lbb, sem).wait()

    def process(sl):
      inb, lbb, _ = bufs[sl]

      @plsc.parallel_loop(0, CHUNK // LANES, unroll=UNROLL)
      def _(i):
        off = i * LANES
        lab = lbb[pl.ds(off, LANES)]
        val = inb[pl.ds(off, LANES)]
        plsc.addupdate_scatter(acc, [row_base + lab], val)

    start(0, 0)
    start(1, 1)

    def outer_body(g2, carry):
      drain(0)
      process(0)
      start(2 * g2 + 2, 0)
      drain(1)
      process(1)
      start(2 * g2 + 3, 1)
      return carry

    lax.fori_loop(0, NCHUNKS // 2 - 1, outer_body, 0)
    drain(0)
    process(0)
    drain(1)
    process(1)

    def reduce_body(cb, carry):
      s = zeros16
      for r in range(LANES):
        s = s + acc[pl.ds(r * ACC_STRIDE + cb * LANES, LANES)]
      part[pl.ds(cb * LANES, LANES)] = s
      return carry

    lax.fori_loop(0, SEG_PAD // LANES, reduce_body, 0)
    pltpu.sync_copy(part, out_hbm.at[wid])

  return seg_sum


_SEG_SUM = _make_sc_kernel()


@jax.jit
def kernel(inputs, labels):
  partials = _SEG_SUM(inputs, labels)
  return jnp.sum(partials, axis=0)[:NSEG]
